# baseline (speedup 1.0000x reference)
import functools
import math

import jax
import jax.numpy as jnp
from jax import lax
from jax.experimental import pallas as pl
from jax.experimental.pallas import tpu as pltpu
from jax.experimental.pallas import tpu_sc as plsc

N = 10000
E = 320000
H = 128
HH = H // 2
NC = 2
NS = 16
L = 16
NW = NC * NS
EPW = E // NW
CH = 80
NCH = EPW // CH
EPT = E // NS
NCH3 = EPT // CH
NP = 5
CPP = NCH3 // NP
DUMP = 1000
BN = 1000
BN1 = 2000

_PERM = [g * 32 + (k // 2 if k % 2 == 0 else 16 + k // 2)
         for g in range(HH // 32) for k in range(32)]

_mesh = plsc.VectorSubcoreMesh(core_axis_name="c", subcore_axis_name="s")
_sc_params = pltpu.CompilerParams(needs_layout_passes=False,
                                  use_tc_tiling_on_sc=False)


def _dense_body(x_ref, wt_ref, a2_ref, h_ref, s_ref):
    xb = x_ref[...]
    hb = xb + jnp.dot(xb, wt_ref[...], preferred_element_type=jnp.float32)
    h_ref[0] = hb[:, :HH].astype(jnp.bfloat16)
    h_ref[1] = hb[:, HH:].astype(jnp.bfloat16)
    s_ref[...] = jnp.dot(xb, a2_ref[...], preferred_element_type=jnp.float32)


def _dense(x, wt, a2):
    return pl.pallas_call(
        _dense_body,
        grid=(N // BN1,),
        in_specs=[
            pl.BlockSpec((BN1, H), lambda i: (i, 0)),
            pl.BlockSpec((H, H), lambda i: (0, 0)),
            pl.BlockSpec((H, 2), lambda i: (0, 0)),
        ],
        out_specs=[
            pl.BlockSpec((NC, BN1, HH), lambda i: (0, i, 0)),
            pl.BlockSpec((BN1, 2), lambda i: (i, 0)),
        ],
        out_shape=[
            jax.ShapeDtypeStruct((NC, N, HH), jnp.bfloat16),
            jax.ShapeDtypeStruct((N, 2), jnp.float32),
        ],
    )(x, wt, a2)


@functools.partial(
    pl.kernel,
    out_type=[
        jax.ShapeDtypeStruct((NW, NCH, CH), jnp.float32),
        jax.ShapeDtypeStruct((NC, 1, N), jnp.float32),
    ],
    mesh=_mesh,
    compiler_params=_sc_params,
    scratch_types=[
        pltpu.VMEM((NCH, CH), jnp.int32),
        pltpu.VMEM((NCH, CH), jnp.int32),
        pltpu.VMEM((NCH, CH), jnp.float32),
        pltpu.VMEM((N,), jnp.float32),
        pltpu.VMEM((N,), jnp.float32),
        pltpu.VMEM((1, N), jnp.float32),
        pltpu.VMEM_SHARED((1, N), jnp.float32),
    ],
)
def _edge_scores(dst_hbm, src_hbm, ei_hbm, ej_hbm,
                 eexp_hbm, seg_hbm,
                 dst_v, src_v, eexp_v, ei_v, ej_v, zero_v, seg_sh):
    c = lax.axis_index("c")
    s = lax.axis_index("s")
    wid = s * NC + c
    pltpu.sync_copy(dst_hbm.at[wid], dst_v)
    pltpu.sync_copy(src_hbm.at[wid], src_v)
    pltpu.sync_copy(ei_hbm, ei_v)
    pltpu.sync_copy(ej_hbm, ej_v)

    def _zb(i, carry):
        zero_v[0, pl.ds(i * L, L)] = jnp.zeros((L,), jnp.float32)
        return carry

    lax.fori_loop(0, N // L, _zb, 0)

    @pl.when(s == 0)
    def _():
        pltpu.sync_copy(zero_v, seg_sh)

    plsc.subcore_barrier()

    def _chunk(j, carry):
        for t in range(CH // L):
            k = t * L
            d = dst_v[j, pl.ds(k, L)]
            sr = src_v[j, pl.ds(k, L)]
            e = plsc.load_gather(ei_v, [d]) + plsc.load_gather(ej_v, [sr])
            e = jnp.where(e > 0, e, 0.01 * e)
            eexp_v[j, pl.ds(k, L)] = jnp.exp(e)
        return carry

    lax.fori_loop(0, NCH, _chunk, 0)

    def _scat(j, carry):
        pltpu.sync_copy(eexp_v.at[j], seg_sh.at[0].at[dst_v.at[j]], add=True)
        return carry

    lax.fori_loop(0, NCH, _scat, 0)

    pltpu.sync_copy(eexp_v, eexp_hbm.at[wid])
    plsc.subcore_barrier()

    @pl.when(s == 0)
    def _():
        pltpu.sync_copy(seg_sh, seg_hbm.at[c])


@functools.partial(
    pl.kernel,
    out_type=jax.ShapeDtypeStruct((NC, N, HH), jnp.float32),
    mesh=_mesh,
    compiler_params=_sc_params,
    scratch_types=[
        pltpu.VMEM((CPP, CH), jnp.int32),
        pltpu.VMEM((CPP, CH), jnp.int32),
        pltpu.VMEM((CPP, CH), jnp.float32),
        pltpu.VMEM((1, N), jnp.float32),
        pltpu.VMEM((1, N), jnp.float32),
        pltpu.VMEM((2, CH, HH), jnp.bfloat16),
        pltpu.VMEM((2, CH, HH), jnp.float32),
        pltpu.VMEM_SHARED((N, HH), jnp.float32),
        pltpu.SemaphoreType.DMA,
        pltpu.SemaphoreType.DMA,
        pltpu.SemaphoreType.DMA,
        pltpu.SemaphoreType.DMA,
    ],
)
def _aggregate(dst_hbm, src_hbm, eexp_hbm, seg_hbm, h_hbm, zeros_hbm,
               out_hbm,
               dst_v, src_v, eexp_v, p0_v, p1_v, rows_v, frows_v, out_sh,
               g0, g1, s0, s1):
    c = lax.axis_index("c")
    s = lax.axis_index("s")
    pltpu.sync_copy(seg_hbm.at[0], p0_v)
    pltpu.sync_copy(seg_hbm.at[1], p1_v)
    @pl.when(s < N // DUMP)
    def _():
        pltpu.sync_copy(zeros_hbm.at[pl.ds(s * DUMP, DUMP)],
                        out_sh.at[pl.ds(s * DUMP, DUMP)])

    zidx = jnp.zeros((L,), jnp.int32)
    hc = h_hbm.at[c]

    def _scale_chunk(j, b):
        rows = rows_v.at[b]
        frows = frows_v.at[b]

        def _scale(eo, inner):
            for u in range(5):
                e = eo * 5 + u
                a = plsc.load_gather(
                    eexp_v,
                    [jnp.full((L,), j, jnp.int32), jnp.full((L,), e, jnp.int32)],
                )
                for g in range(HH // 32):
                    xb = rows[e, pl.ds(g * 32, 32)]
                    lo, hi = plsc.unpack(xb, format=plsc.PackFormat.INTERLEAVED)
                    frows[e, pl.ds(g * 32, L)] = lo * a
                    frows[e, pl.ds(g * 32 + L, L)] = hi * a
            return inner

        lax.fori_loop(0, CH // 5, _scale, 0)

    def _pair(jj, carry):
        j0 = jj * 2
        j1 = j0 + 1
        pltpu.make_async_copy(hc.at[dst_v.at[j0]], rows_v.at[0], g0).wait()
        _scale_chunk(j0, 0)
        s0d = pltpu.async_copy(frows_v.at[0], out_sh.at[src_v.at[j0]], s0,
                               add=True)

        @pl.when(jj + 1 < CPP // 2)
        def _():
            pltpu.async_copy(hc.at[dst_v.at[j0 + 2]], rows_v.at[0], g0)

        pltpu.make_async_copy(hc.at[dst_v.at[j1]], rows_v.at[1], g1).wait()
        _scale_chunk(j1, 1)
        s1d = pltpu.async_copy(frows_v.at[1], out_sh.at[src_v.at[j1]], s1,
                               add=True)

        @pl.when(jj + 1 < CPP // 2)
        def _():
            pltpu.async_copy(hc.at[dst_v.at[j1 + 2]], rows_v.at[1], g1)

        s0d.wait()
        s1d.wait()
        return carry

    for p in range(NP):
        tid = s * NP + p
        pltpu.sync_copy(dst_hbm.at[tid], dst_v)
        pltpu.sync_copy(src_hbm.at[tid], src_v)
        pltpu.sync_copy(eexp_hbm.at[tid], eexp_v)

        def _al(j, carry):
            for t in range(CH // L):
                k = t * L
                d = dst_v[j, pl.ds(k, L)]
                ssum = (plsc.load_gather(p0_v, [zidx, d])
                        + plsc.load_gather(p1_v, [zidx, d]))
                eexp_v[j, pl.ds(k, L)] = eexp_v[j, pl.ds(k, L)] / (ssum + 1e-16)
            return carry

        lax.fori_loop(0, CPP, _al, 0)
        plsc.subcore_barrier()
        pltpu.async_copy(hc.at[dst_v.at[0]], rows_v.at[0], g0)
        pltpu.async_copy(hc.at[dst_v.at[1]], rows_v.at[1], g1)
        lax.fori_loop(0, CPP // 2, _pair, 0)
    plsc.subcore_barrier()

    @pl.when(s < N // DUMP)
    def _():
        pltpu.sync_copy(out_sh.at[pl.ds(s * DUMP, DUMP)],
                        out_hbm.at[c, pl.ds(s * DUMP, DUMP)])


_INV_SQRT2 = 1.0 / math.sqrt(2.0)


def _finish_body(p_ref, o_ref):
    a = p_ref[0]
    o_ref[:, :HH] = a * 0.5 * (1.0 + lax.erf(a * _INV_SQRT2))
    b = p_ref[1]
    o_ref[:, HH:] = b * 0.5 * (1.0 + lax.erf(b * _INV_SQRT2))


def _finish(parts):
    return pl.pallas_call(
        _finish_body,
        grid=(N // BN,),
        in_specs=[pl.BlockSpec((NC, BN, HH), lambda i: (0, i, 0))],
        out_specs=pl.BlockSpec((BN, H), lambda i: (i, 0)),
        out_shape=jax.ShapeDtypeStruct((N, H), jnp.float32),
    )(parts)


@jax.jit
def _impl(x, edge_index, a_i, a_j, Wx):
    wt = Wx.T
    a2 = jnp.stack([a_i, a_j], axis=1)
    h2, scores = _dense(x, wt, a2)
    h2p = h2[:, :, jnp.array(_PERM, jnp.int32)]
    ei = scores[:, 0]
    ej = scores[:, 1]
    src = edge_index[0].reshape(NW, NCH, CH)
    dst = edge_index[1].reshape(NW, NCH, CH)
    eexp, seg = _edge_scores(dst, src, ei, ej)
    zeros = jnp.zeros((N, HH), jnp.float32)
    parts = _aggregate(dst.reshape(NS * NP, CPP, CH),
                       src.reshape(NS * NP, CPP, CH),
                       eexp.reshape(NS * NP, CPP, CH), seg, h2p, zeros)
    return _finish(parts)


def kernel(x, edge_index, a_i, a_j, Wx):
    return _impl(x, edge_index, a_i, a_j, Wx)


# speedup vs baseline: 1.0143x; 1.0143x over previous
import functools
import math

import jax
import jax.numpy as jnp
from jax import lax
from jax.experimental import pallas as pl
from jax.experimental.pallas import tpu as pltpu
from jax.experimental.pallas import tpu_sc as plsc

N = 10000
E = 320000
H = 128
HH = H // 2
NC = 2
NS = 16
L = 16
NW = NC * NS
EPW = E // NW
CH = 80
NCH = EPW // CH
EPT = E // NS
NCH3 = EPT // CH
NP = 5
CPP = NCH3 // NP
DUMP = 1000
BN = 1000
BN1 = 2000

_PERM = [g * 32 + (k // 2 if k % 2 == 0 else 16 + k // 2)
         for g in range(HH // 32) for k in range(32)]

_mesh = plsc.VectorSubcoreMesh(core_axis_name="c", subcore_axis_name="s")
_sc_params = pltpu.CompilerParams(needs_layout_passes=False,
                                  use_tc_tiling_on_sc=False)


def _dense_body(x_ref, wt_ref, a2_ref, h_ref, s_ref):
    xb = x_ref[...]
    hb = xb + jnp.dot(xb, wt_ref[...], preferred_element_type=jnp.float32)
    h_ref[0] = hb[:, :HH].astype(jnp.bfloat16)
    h_ref[1] = hb[:, HH:].astype(jnp.bfloat16)
    s_ref[...] = jnp.dot(xb, a2_ref[...], preferred_element_type=jnp.float32)


def _dense(x, wt, a2):
    return pl.pallas_call(
        _dense_body,
        grid=(N // BN1,),
        in_specs=[
            pl.BlockSpec((BN1, H), lambda i: (i, 0)),
            pl.BlockSpec((H, H), lambda i: (0, 0)),
            pl.BlockSpec((H, 2), lambda i: (0, 0)),
        ],
        out_specs=[
            pl.BlockSpec((NC, BN1, HH), lambda i: (0, i, 0)),
            pl.BlockSpec((BN1, 2), lambda i: (i, 0)),
        ],
        out_shape=[
            jax.ShapeDtypeStruct((NC, N, HH), jnp.bfloat16),
            jax.ShapeDtypeStruct((N, 2), jnp.float32),
        ],
    )(x, wt, a2)


@functools.partial(
    pl.kernel,
    out_type=[
        jax.ShapeDtypeStruct((NW, NCH, CH), jnp.float32),
        jax.ShapeDtypeStruct((NC, 1, N), jnp.float32),
    ],
    mesh=_mesh,
    compiler_params=_sc_params,
    scratch_types=[
        pltpu.VMEM((NCH, CH), jnp.int32),
        pltpu.VMEM((NCH, CH), jnp.int32),
        pltpu.VMEM((NCH, CH), jnp.float32),
        pltpu.VMEM((N,), jnp.float32),
        pltpu.VMEM((N,), jnp.float32),
        pltpu.VMEM((1, N), jnp.float32),
        pltpu.VMEM_SHARED((1, N), jnp.float32),
    ],
)
def _edge_scores(dst_hbm, src_hbm, ei_hbm, ej_hbm,
                 eexp_hbm, seg_hbm,
                 dst_v, src_v, eexp_v, ei_v, ej_v, zero_v, seg_sh):
    c = lax.axis_index("c")
    s = lax.axis_index("s")
    wid = s * NC + c
    pltpu.sync_copy(dst_hbm.at[wid], dst_v)
    pltpu.sync_copy(src_hbm.at[wid], src_v)
    pltpu.sync_copy(ei_hbm, ei_v)
    pltpu.sync_copy(ej_hbm, ej_v)

    def _zb(i, carry):
        zero_v[0, pl.ds(i * L, L)] = jnp.zeros((L,), jnp.float32)
        return carry

    lax.fori_loop(0, N // L, _zb, 0)

    @pl.when(s == 0)
    def _():
        pltpu.sync_copy(zero_v, seg_sh)

    plsc.subcore_barrier()

    def _chunk(j, carry):
        for t in range(CH // L):
            k = t * L
            d = dst_v[j, pl.ds(k, L)]
            sr = src_v[j, pl.ds(k, L)]
            e = plsc.load_gather(ei_v, [d]) + plsc.load_gather(ej_v, [sr])
            e = jnp.where(e > 0, e, 0.01 * e)
            eexp_v[j, pl.ds(k, L)] = jnp.exp(e)
        return carry

    lax.fori_loop(0, NCH, _chunk, 0)

    def _scat(j, carry):
        pltpu.sync_copy(eexp_v.at[j], seg_sh.at[0].at[dst_v.at[j]], add=True)
        return carry

    lax.fori_loop(0, NCH, _scat, 0)

    pltpu.sync_copy(eexp_v, eexp_hbm.at[wid])
    plsc.subcore_barrier()

    @pl.when(s == 0)
    def _():
        pltpu.sync_copy(seg_sh, seg_hbm.at[c])


@functools.partial(
    pl.kernel,
    out_type=jax.ShapeDtypeStruct((NC, N, HH), jnp.float32),
    mesh=_mesh,
    compiler_params=_sc_params,
    scratch_types=[
        pltpu.VMEM((CPP, CH), jnp.int32),
        pltpu.VMEM((CPP, CH), jnp.int32),
        pltpu.VMEM((CPP, CH), jnp.float32),
        pltpu.VMEM((1, N), jnp.float32),
        pltpu.VMEM((1, N), jnp.float32),
        pltpu.VMEM((2, CH, HH), jnp.bfloat16),
        pltpu.VMEM((2, CH, HH), jnp.float32),
        pltpu.VMEM_SHARED((N, HH), jnp.float32),
        pltpu.SemaphoreType.DMA,
        pltpu.SemaphoreType.DMA,
        pltpu.SemaphoreType.DMA,
        pltpu.SemaphoreType.DMA,
    ],
)
def _aggregate(dst_hbm, src_hbm, eexp_hbm, seg_hbm, h_hbm, zeros_hbm,
               out_hbm,
               dst_v, src_v, eexp_v, p0_v, p1_v, rows_v, frows_v, out_sh,
               g0, g1, s0, s1):
    c = lax.axis_index("c")
    s = lax.axis_index("s")
    pltpu.sync_copy(seg_hbm.at[0], p0_v)
    pltpu.sync_copy(seg_hbm.at[1], p1_v)
    @pl.when(s < N // DUMP)
    def _():
        pltpu.sync_copy(zeros_hbm.at[pl.ds(s * DUMP, DUMP)],
                        out_sh.at[pl.ds(s * DUMP, DUMP)])

    zidx = jnp.zeros((L,), jnp.int32)
    hc = h_hbm.at[c]

    def _scale_chunk(j, b):
        rows = rows_v.at[b]
        frows = frows_v.at[b]
        jv = jnp.full((L,), j, jnp.int32)

        def _scale(eo, inner):
            base = eo * L
            basev = jnp.full((L,), base, jnp.int32)
            for u in range(L):
                e = base + u
                a = plsc.load_gather(eexp_v, [jv, basev + u])
                for g in range(HH // 32):
                    xb = rows[e, pl.ds(g * 32, 32)]
                    lo, hi = plsc.unpack(xb, format=plsc.PackFormat.INTERLEAVED)
                    frows[e, pl.ds(g * 32, L)] = lo * a
                    frows[e, pl.ds(g * 32 + L, L)] = hi * a
            return inner

        lax.fori_loop(0, CH // L, _scale, 0)

    def _pair(jj, carry):
        j0 = jj * 2
        j1 = j0 + 1
        pltpu.make_async_copy(hc.at[dst_v.at[j0]], rows_v.at[0], g0).wait()
        _scale_chunk(j0, 0)
        s0d = pltpu.async_copy(frows_v.at[0], out_sh.at[src_v.at[j0]], s0,
                               add=True)

        @pl.when(jj + 1 < CPP // 2)
        def _():
            pltpu.async_copy(hc.at[dst_v.at[j0 + 2]], rows_v.at[0], g0)

        pltpu.make_async_copy(hc.at[dst_v.at[j1]], rows_v.at[1], g1).wait()
        _scale_chunk(j1, 1)
        s1d = pltpu.async_copy(frows_v.at[1], out_sh.at[src_v.at[j1]], s1,
                               add=True)

        @pl.when(jj + 1 < CPP // 2)
        def _():
            pltpu.async_copy(hc.at[dst_v.at[j1 + 2]], rows_v.at[1], g1)

        s0d.wait()
        s1d.wait()
        return carry

    for p in range(NP):
        tid = s * NP + p
        pltpu.sync_copy(dst_hbm.at[tid], dst_v)
        pltpu.sync_copy(src_hbm.at[tid], src_v)
        pltpu.sync_copy(eexp_hbm.at[tid], eexp_v)

        def _al(j, carry):
            for t in range(CH // L):
                k = t * L
                d = dst_v[j, pl.ds(k, L)]
                ssum = (plsc.load_gather(p0_v, [zidx, d])
                        + plsc.load_gather(p1_v, [zidx, d]))
                eexp_v[j, pl.ds(k, L)] = eexp_v[j, pl.ds(k, L)] / (ssum + 1e-16)
            return carry

        lax.fori_loop(0, CPP, _al, 0)
        plsc.subcore_barrier()
        pltpu.async_copy(hc.at[dst_v.at[0]], rows_v.at[0], g0)
        pltpu.async_copy(hc.at[dst_v.at[1]], rows_v.at[1], g1)
        lax.fori_loop(0, CPP // 2, _pair, 0)
    plsc.subcore_barrier()

    @pl.when(s < N // DUMP)
    def _():
        pltpu.sync_copy(out_sh.at[pl.ds(s * DUMP, DUMP)],
                        out_hbm.at[c, pl.ds(s * DUMP, DUMP)])


_INV_SQRT2 = 1.0 / math.sqrt(2.0)


def _finish_body(p_ref, o_ref):
    a = p_ref[0]
    o_ref[:, :HH] = a * 0.5 * (1.0 + lax.erf(a * _INV_SQRT2))
    b = p_ref[1]
    o_ref[:, HH:] = b * 0.5 * (1.0 + lax.erf(b * _INV_SQRT2))


def _finish(parts):
    return pl.pallas_call(
        _finish_body,
        grid=(N // BN,),
        in_specs=[pl.BlockSpec((NC, BN, HH), lambda i: (0, i, 0))],
        out_specs=pl.BlockSpec((BN, H), lambda i: (i, 0)),
        out_shape=jax.ShapeDtypeStruct((N, H), jnp.float32),
    )(parts)


@jax.jit
def _impl(x, edge_index, a_i, a_j, Wx):
    wt = Wx.T
    a2 = jnp.stack([a_i, a_j], axis=1)
    h2, scores = _dense(x, wt, a2)
    h2p = h2[:, :, jnp.array(_PERM, jnp.int32)]
    ei = scores[:, 0]
    ej = scores[:, 1]
    src = edge_index[0].reshape(NW, NCH, CH)
    dst = edge_index[1].reshape(NW, NCH, CH)
    eexp, seg = _edge_scores(dst, src, ei, ej)
    zeros = jnp.zeros((N, HH), jnp.float32)
    parts = _aggregate(dst.reshape(NS * NP, CPP, CH),
                       src.reshape(NS * NP, CPP, CH),
                       eexp.reshape(NS * NP, CPP, CH), seg, h2p, zeros)
    return _finish(parts)


def kernel(x, edge_index, a_i, a_j, Wx):
    return _impl(x, edge_index, a_i, a_j, Wx)


# speedup vs baseline: 1.6546x; 1.6313x over previous
import functools
import math

import jax
import jax.numpy as jnp
from jax import lax
from jax.experimental import pallas as pl
from jax.experimental.pallas import tpu as pltpu
from jax.experimental.pallas import tpu_sc as plsc

N = 10000
E = 320000
H = 128
HH = H // 2
NC = 2
NS = 16
L = 16
NW = NC * NS
EPW = E // NW
CH = 80
NCH = EPW // CH
EPT = E // NS
NCH3 = EPT // CH
NP = 5
CPP = NCH3 // NP
DUMP = 1000
BN = 1000
BN1 = 2000

_PERM = [g * 32 + (k // 2 if k % 2 == 0 else 16 + k // 2)
         for g in range(HH // 32) for k in range(32)]

_mesh = plsc.VectorSubcoreMesh(core_axis_name="c", subcore_axis_name="s")
_sc_params = pltpu.CompilerParams(needs_layout_passes=False,
                                  use_tc_tiling_on_sc=False)


def _dense_body(x_ref, wt_ref, a2_ref, h_ref, s_ref):
    xb = x_ref[...]
    hb = xb + jnp.dot(xb, wt_ref[...], preferred_element_type=jnp.float32)
    h_ref[0] = hb[:, :HH].astype(jnp.bfloat16)
    h_ref[1] = hb[:, HH:].astype(jnp.bfloat16)
    s_ref[...] = jnp.dot(xb, a2_ref[...], preferred_element_type=jnp.float32)


def _dense(x, wt, a2):
    return pl.pallas_call(
        _dense_body,
        grid=(N // BN1,),
        in_specs=[
            pl.BlockSpec((BN1, H), lambda i: (i, 0)),
            pl.BlockSpec((H, H), lambda i: (0, 0)),
            pl.BlockSpec((H, 2), lambda i: (0, 0)),
        ],
        out_specs=[
            pl.BlockSpec((NC, BN1, HH), lambda i: (0, i, 0)),
            pl.BlockSpec((BN1, 2), lambda i: (i, 0)),
        ],
        out_shape=[
            jax.ShapeDtypeStruct((NC, N, HH), jnp.bfloat16),
            jax.ShapeDtypeStruct((N, 2), jnp.float32),
        ],
    )(x, wt, a2)


@functools.partial(
    pl.kernel,
    out_type=[
        jax.ShapeDtypeStruct((NW, NCH, CH), jnp.float32),
        jax.ShapeDtypeStruct((NC, 1, N), jnp.float32),
    ],
    mesh=_mesh,
    compiler_params=_sc_params,
    scratch_types=[
        pltpu.VMEM((NCH, CH), jnp.int32),
        pltpu.VMEM((NCH, CH), jnp.int32),
        pltpu.VMEM((NCH, CH), jnp.float32),
        pltpu.VMEM((N,), jnp.float32),
        pltpu.VMEM((N,), jnp.float32),
        pltpu.VMEM((1, N), jnp.float32),
        pltpu.VMEM_SHARED((1, N), jnp.float32),
    ],
)
def _edge_scores(dst_hbm, src_hbm, ei_hbm, ej_hbm,
                 eexp_hbm, seg_hbm,
                 dst_v, src_v, eexp_v, ei_v, ej_v, zero_v, seg_sh):
    c = lax.axis_index("c")
    s = lax.axis_index("s")
    wid = s * NC + c
    pltpu.sync_copy(dst_hbm.at[wid], dst_v)
    pltpu.sync_copy(src_hbm.at[wid], src_v)
    pltpu.sync_copy(ei_hbm, ei_v)
    pltpu.sync_copy(ej_hbm, ej_v)

    def _zb(i, carry):
        zero_v[0, pl.ds(i * L, L)] = jnp.zeros((L,), jnp.float32)
        return carry

    lax.fori_loop(0, N // L, _zb, 0)

    @pl.when(s == 0)
    def _():
        pltpu.sync_copy(zero_v, seg_sh)

    plsc.subcore_barrier()

    @plsc.parallel_loop(0, NCH, 1, unroll=2)
    def _chunk(j):
        for t in range(CH // L):
            k = t * L
            d = dst_v[j, pl.ds(k, L)]
            sr = src_v[j, pl.ds(k, L)]
            e = plsc.load_gather(ei_v, [d]) + plsc.load_gather(ej_v, [sr])
            e = jnp.where(e > 0, e, 0.01 * e)
            eexp_v[j, pl.ds(k, L)] = jnp.exp(e)

    def _scat(j, carry):
        pltpu.sync_copy(eexp_v.at[j], seg_sh.at[0].at[dst_v.at[j]], add=True)
        return carry

    lax.fori_loop(0, NCH, _scat, 0)

    pltpu.sync_copy(eexp_v, eexp_hbm.at[wid])
    plsc.subcore_barrier()

    @pl.when(s == 0)
    def _():
        pltpu.sync_copy(seg_sh, seg_hbm.at[c])


@functools.partial(
    pl.kernel,
    out_type=jax.ShapeDtypeStruct((NC, N, HH), jnp.float32),
    mesh=_mesh,
    compiler_params=_sc_params,
    scratch_types=[
        pltpu.VMEM((CPP, CH), jnp.int32),
        pltpu.VMEM((CPP, CH), jnp.int32),
        pltpu.VMEM((CPP, CH), jnp.float32),
        pltpu.VMEM((1, N), jnp.float32),
        pltpu.VMEM((1, N), jnp.float32),
        pltpu.VMEM((2, CH, HH), jnp.bfloat16),
        pltpu.VMEM((2, CH, HH), jnp.float32),
        pltpu.VMEM_SHARED((N, HH), jnp.float32),
        pltpu.SemaphoreType.DMA,
        pltpu.SemaphoreType.DMA,
        pltpu.SemaphoreType.DMA,
        pltpu.SemaphoreType.DMA,
    ],
)
def _aggregate(dst_hbm, src_hbm, eexp_hbm, seg_hbm, h_hbm, zeros_hbm,
               out_hbm,
               dst_v, src_v, eexp_v, p0_v, p1_v, rows_v, frows_v, out_sh,
               g0, g1, s0, s1):
    c = lax.axis_index("c")
    s = lax.axis_index("s")
    pltpu.sync_copy(seg_hbm.at[0], p0_v)
    pltpu.sync_copy(seg_hbm.at[1], p1_v)
    @pl.when(s < N // DUMP)
    def _():
        pltpu.sync_copy(zeros_hbm.at[pl.ds(s * DUMP, DUMP)],
                        out_sh.at[pl.ds(s * DUMP, DUMP)])

    zidx = jnp.zeros((L,), jnp.int32)
    hc = h_hbm.at[c]

    def _scale_chunk(j, b):
        rows = rows_v.at[b]
        frows = frows_v.at[b]
        jv = jnp.full((L,), j, jnp.int32)

        @plsc.parallel_loop(0, CH, 1, unroll=8)
        def _scale(e):
            a = plsc.load_gather(eexp_v, [jv, jnp.full((L,), e, jnp.int32)])
            for g in range(HH // 32):
                xb = rows[e, pl.ds(g * 32, 32)]
                lo, hi = plsc.unpack(xb, format=plsc.PackFormat.INTERLEAVED)
                frows[e, pl.ds(g * 32, L)] = lo * a
                frows[e, pl.ds(g * 32 + L, L)] = hi * a

    def _pair(jj, carry):
        j0 = jj * 2
        j1 = j0 + 1
        pltpu.make_async_copy(hc.at[dst_v.at[j0]], rows_v.at[0], g0).wait()
        _scale_chunk(j0, 0)
        s0d = pltpu.async_copy(frows_v.at[0], out_sh.at[src_v.at[j0]], s0,
                               add=True)

        @pl.when(jj + 1 < CPP // 2)
        def _():
            pltpu.async_copy(hc.at[dst_v.at[j0 + 2]], rows_v.at[0], g0)

        pltpu.make_async_copy(hc.at[dst_v.at[j1]], rows_v.at[1], g1).wait()
        _scale_chunk(j1, 1)
        s1d = pltpu.async_copy(frows_v.at[1], out_sh.at[src_v.at[j1]], s1,
                               add=True)

        @pl.when(jj + 1 < CPP // 2)
        def _():
            pltpu.async_copy(hc.at[dst_v.at[j1 + 2]], rows_v.at[1], g1)

        s0d.wait()
        s1d.wait()
        return carry

    for p in range(NP):
        tid = s * NP + p
        pltpu.sync_copy(dst_hbm.at[tid], dst_v)
        pltpu.sync_copy(src_hbm.at[tid], src_v)
        pltpu.sync_copy(eexp_hbm.at[tid], eexp_v)

        @plsc.parallel_loop(0, CPP, 1, unroll=2)
        def _al(j):
            for t in range(CH // L):
                k = t * L
                d = dst_v[j, pl.ds(k, L)]
                ssum = (plsc.load_gather(p0_v, [zidx, d])
                        + plsc.load_gather(p1_v, [zidx, d]))
                eexp_v[j, pl.ds(k, L)] = eexp_v[j, pl.ds(k, L)] / (ssum + 1e-16)
        plsc.subcore_barrier()
        pltpu.async_copy(hc.at[dst_v.at[0]], rows_v.at[0], g0)
        pltpu.async_copy(hc.at[dst_v.at[1]], rows_v.at[1], g1)
        lax.fori_loop(0, CPP // 2, _pair, 0)
    plsc.subcore_barrier()

    @pl.when(s < N // DUMP)
    def _():
        pltpu.sync_copy(out_sh.at[pl.ds(s * DUMP, DUMP)],
                        out_hbm.at[c, pl.ds(s * DUMP, DUMP)])


_INV_SQRT2 = 1.0 / math.sqrt(2.0)


def _finish_body(p_ref, o_ref):
    a = p_ref[0]
    o_ref[:, :HH] = a * 0.5 * (1.0 + lax.erf(a * _INV_SQRT2))
    b = p_ref[1]
    o_ref[:, HH:] = b * 0.5 * (1.0 + lax.erf(b * _INV_SQRT2))


def _finish(parts):
    return pl.pallas_call(
        _finish_body,
        grid=(N // BN,),
        in_specs=[pl.BlockSpec((NC, BN, HH), lambda i: (0, i, 0))],
        out_specs=pl.BlockSpec((BN, H), lambda i: (i, 0)),
        out_shape=jax.ShapeDtypeStruct((N, H), jnp.float32),
    )(parts)


@jax.jit
def _impl(x, edge_index, a_i, a_j, Wx):
    wt = Wx.T
    a2 = jnp.stack([a_i, a_j], axis=1)
    h2, scores = _dense(x, wt, a2)
    h2p = h2[:, :, jnp.array(_PERM, jnp.int32)]
    ei = scores[:, 0]
    ej = scores[:, 1]
    src = edge_index[0].reshape(NW, NCH, CH)
    dst = edge_index[1].reshape(NW, NCH, CH)
    eexp, seg = _edge_scores(dst, src, ei, ej)
    zeros = jnp.zeros((N, HH), jnp.float32)
    parts = _aggregate(dst.reshape(NS * NP, CPP, CH),
                       src.reshape(NS * NP, CPP, CH),
                       eexp.reshape(NS * NP, CPP, CH), seg, h2p, zeros)
    return _finish(parts)


def kernel(x, edge_index, a_i, a_j, Wx):
    return _impl(x, edge_index, a_i, a_j, Wx)


# speedup vs baseline: 1.7349x; 1.0485x over previous
import functools
import math

import jax
import jax.numpy as jnp
from jax import lax
from jax.experimental import pallas as pl
from jax.experimental.pallas import tpu as pltpu
from jax.experimental.pallas import tpu_sc as plsc

N = 10000
E = 320000
H = 128
HH = H // 2
NC = 2
NS = 16
L = 16
NW = NC * NS
EPW = E // NW
CH = 80
NCH = EPW // CH
EPT = E // NS
NCH3 = EPT // CH
NP = 5
CPP = NCH3 // NP
DUMP = 1000
BN = 1000
BN1 = 2000

_PERM = [g * 32 + (k // 2 if k % 2 == 0 else 16 + k // 2)
         for g in range(HH // 32) for k in range(32)]

_mesh = plsc.VectorSubcoreMesh(core_axis_name="c", subcore_axis_name="s")
_sc_params = pltpu.CompilerParams(needs_layout_passes=False,
                                  use_tc_tiling_on_sc=False)


def _dense_body(x_ref, m_ref, a2_ref, h_ref, s_ref):
    xb = x_ref[...]
    hb = jnp.dot(xb, m_ref[...], preferred_element_type=jnp.float32)
    h_ref[0] = hb[:, :HH].astype(jnp.bfloat16)
    h_ref[1] = hb[:, HH:].astype(jnp.bfloat16)
    s_ref[...] = jnp.dot(xb, a2_ref[...], preferred_element_type=jnp.float32)


def _dense(x, m, a2):
    return pl.pallas_call(
        _dense_body,
        grid=(N // BN1,),
        in_specs=[
            pl.BlockSpec((BN1, H), lambda i: (i, 0)),
            pl.BlockSpec((H, H), lambda i: (0, 0)),
            pl.BlockSpec((H, 2), lambda i: (0, 0)),
        ],
        out_specs=[
            pl.BlockSpec((NC, BN1, HH), lambda i: (0, i, 0)),
            pl.BlockSpec((BN1, 2), lambda i: (i, 0)),
        ],
        out_shape=[
            jax.ShapeDtypeStruct((NC, N, HH), jnp.bfloat16),
            jax.ShapeDtypeStruct((N, 2), jnp.float32),
        ],
    )(x, m, a2)


@functools.partial(
    pl.kernel,
    out_type=[
        jax.ShapeDtypeStruct((NW, NCH, CH), jnp.float32),
        jax.ShapeDtypeStruct((NC, 1, N), jnp.float32),
    ],
    mesh=_mesh,
    compiler_params=_sc_params,
    scratch_types=[
        pltpu.VMEM((NCH, CH), jnp.int32),
        pltpu.VMEM((NCH, CH), jnp.int32),
        pltpu.VMEM((NCH, CH), jnp.float32),
        pltpu.VMEM((N,), jnp.float32),
        pltpu.VMEM((N,), jnp.float32),
        pltpu.VMEM((1, N), jnp.float32),
        pltpu.VMEM_SHARED((1, N), jnp.float32),
        pltpu.SemaphoreType.DMA,
    ],
)
def _edge_scores(dst_hbm, src_hbm, ei_hbm, ej_hbm,
                 eexp_hbm, seg_hbm,
                 dst_v, src_v, eexp_v, ei_v, ej_v, zero_v, seg_sh, ssem):
    c = lax.axis_index("c")
    s = lax.axis_index("s")
    wid = s * NC + c
    pltpu.sync_copy(dst_hbm.at[wid], dst_v)
    pltpu.sync_copy(src_hbm.at[wid], src_v)
    pltpu.sync_copy(ei_hbm, ei_v)
    pltpu.sync_copy(ej_hbm, ej_v)

    def _zb(i, carry):
        zero_v[0, pl.ds(i * L, L)] = jnp.zeros((L,), jnp.float32)
        return carry

    lax.fori_loop(0, N // L, _zb, 0)

    @pl.when(s == 0)
    def _():
        pltpu.sync_copy(zero_v, seg_sh)

    plsc.subcore_barrier()

    @plsc.parallel_loop(0, NCH, 1, unroll=2)
    def _chunk(j):
        for t in range(CH // L):
            k = t * L
            d = dst_v[j, pl.ds(k, L)]
            sr = src_v[j, pl.ds(k, L)]
            e = plsc.load_gather(ei_v, [d]) + plsc.load_gather(ej_v, [sr])
            e = jnp.where(e > 0, e, 0.01 * e)
            eexp_v[j, pl.ds(k, L)] = jnp.exp(e)

    def _scat(j, carry):
        pltpu.async_copy(eexp_v.at[j], seg_sh.at[0].at[dst_v.at[j]], ssem,
                         add=True)
        return carry

    lax.fori_loop(0, NCH, _scat, 0)

    def _drain(j, carry):
        pltpu.make_async_copy(eexp_v.at[j], seg_sh.at[0].at[dst_v.at[j]],
                              ssem).wait()
        return carry

    lax.fori_loop(0, NCH, _drain, 0)

    pltpu.sync_copy(eexp_v, eexp_hbm.at[wid])
    plsc.subcore_barrier()

    @pl.when(s == 0)
    def _():
        pltpu.sync_copy(seg_sh, seg_hbm.at[c])


@functools.partial(
    pl.kernel,
    out_type=jax.ShapeDtypeStruct((NC, N, HH), jnp.float32),
    mesh=_mesh,
    compiler_params=_sc_params,
    scratch_types=[
        pltpu.VMEM((CPP, CH), jnp.int32),
        pltpu.VMEM((CPP, CH), jnp.int32),
        pltpu.VMEM((CPP, CH), jnp.float32),
        pltpu.VMEM((1, N), jnp.float32),
        pltpu.VMEM((1, N), jnp.float32),
        pltpu.VMEM((2, CH, HH), jnp.bfloat16),
        pltpu.VMEM((2, CH, HH), jnp.float32),
        pltpu.VMEM((200, HH), jnp.float32),
        pltpu.VMEM_SHARED((N, HH), jnp.float32),
        pltpu.SemaphoreType.DMA,
        pltpu.SemaphoreType.DMA,
        pltpu.SemaphoreType.DMA,
        pltpu.SemaphoreType.DMA,
    ],
)
def _aggregate(dst_hbm, src_hbm, eexp_hbm, seg_hbm, h_hbm,
               out_hbm,
               dst_v, src_v, eexp_v, p0_v, p1_v, rows_v, frows_v, zblk_v,
               out_sh, g0, g1, s0, s1):
    c = lax.axis_index("c")
    s = lax.axis_index("s")
    pltpu.sync_copy(seg_hbm.at[0], p0_v)
    pltpu.sync_copy(seg_hbm.at[1], p1_v)

    @plsc.parallel_loop(0, 200, 1, unroll=2)
    def _zb(i):
        for t in range(HH // L):
            zblk_v[i, pl.ds(t * L, L)] = jnp.zeros((L,), jnp.float32)

    @pl.when(s < N // DUMP)
    def _():
        for q in range(DUMP // 200):
            pltpu.sync_copy(zblk_v,
                            out_sh.at[pl.ds(s * DUMP + q * 200, 200)])

    zidx = jnp.zeros((L,), jnp.int32)
    hc = h_hbm.at[c]

    def _scale_chunk(j, b):
        rows = rows_v.at[b]
        frows = frows_v.at[b]
        jv = jnp.full((L,), j, jnp.int32)

        @plsc.parallel_loop(0, CH, 1, unroll=8)
        def _scale(e):
            a = plsc.load_gather(eexp_v, [jv, jnp.full((L,), e, jnp.int32)])
            for g in range(HH // 32):
                xb = rows[e, pl.ds(g * 32, 32)]
                lo, hi = plsc.unpack(xb, format=plsc.PackFormat.INTERLEAVED)
                frows[e, pl.ds(g * 32, L)] = lo * a
                frows[e, pl.ds(g * 32 + L, L)] = hi * a

    def _pair(jj, carry):
        j0 = jj * 2
        j1 = j0 + 1
        pltpu.make_async_copy(hc.at[dst_v.at[j0]], rows_v.at[0], g0).wait()
        _scale_chunk(j0, 0)
        s0d = pltpu.async_copy(frows_v.at[0], out_sh.at[src_v.at[j0]], s0,
                               add=True)

        @pl.when(jj + 1 < CPP // 2)
        def _():
            pltpu.async_copy(hc.at[dst_v.at[j0 + 2]], rows_v.at[0], g0)

        pltpu.make_async_copy(hc.at[dst_v.at[j1]], rows_v.at[1], g1).wait()
        _scale_chunk(j1, 1)
        s1d = pltpu.async_copy(frows_v.at[1], out_sh.at[src_v.at[j1]], s1,
                               add=True)

        @pl.when(jj + 1 < CPP // 2)
        def _():
            pltpu.async_copy(hc.at[dst_v.at[j1 + 2]], rows_v.at[1], g1)

        s0d.wait()
        s1d.wait()
        return carry

    for p in range(NP):
        tid = s * NP + p
        pltpu.sync_copy(dst_hbm.at[tid], dst_v)
        pltpu.sync_copy(src_hbm.at[tid], src_v)
        pltpu.sync_copy(eexp_hbm.at[tid], eexp_v)

        @plsc.parallel_loop(0, CPP, 1, unroll=2)
        def _al(j):
            for t in range(CH // L):
                k = t * L
                d = dst_v[j, pl.ds(k, L)]
                ssum = (plsc.load_gather(p0_v, [zidx, d])
                        + plsc.load_gather(p1_v, [zidx, d]))
                eexp_v[j, pl.ds(k, L)] = eexp_v[j, pl.ds(k, L)] / (ssum + 1e-16)
        plsc.subcore_barrier()
        pltpu.async_copy(hc.at[dst_v.at[0]], rows_v.at[0], g0)
        pltpu.async_copy(hc.at[dst_v.at[1]], rows_v.at[1], g1)
        lax.fori_loop(0, CPP // 2, _pair, 0)
    plsc.subcore_barrier()

    @pl.when(s < N // DUMP)
    def _():
        pltpu.sync_copy(out_sh.at[pl.ds(s * DUMP, DUMP)],
                        out_hbm.at[c, pl.ds(s * DUMP, DUMP)])


_INV_SQRT2 = 1.0 / math.sqrt(2.0)


def _finish_body(p_ref, o_ref):
    a = p_ref[0]
    o_ref[:, :HH] = a * 0.5 * (1.0 + lax.erf(a * _INV_SQRT2))
    b = p_ref[1]
    o_ref[:, HH:] = b * 0.5 * (1.0 + lax.erf(b * _INV_SQRT2))


def _finish(parts):
    return pl.pallas_call(
        _finish_body,
        grid=(N // BN,),
        in_specs=[pl.BlockSpec((NC, BN, HH), lambda i: (0, i, 0))],
        out_specs=pl.BlockSpec((BN, H), lambda i: (i, 0)),
        out_shape=jax.ShapeDtypeStruct((N, H), jnp.float32),
    )(parts)


@jax.jit
def _impl(x, edge_index, a_i, a_j, Wx):
    perm128 = jnp.array(_PERM + [HH + p for p in _PERM], jnp.int32)
    m = (jnp.eye(H, dtype=jnp.float32) + Wx.T)[:, perm128]
    a2 = jnp.stack([a_i, a_j], axis=1)
    h2, scores = _dense(x, m, a2)
    ei = scores[:, 0]
    ej = scores[:, 1]
    src = edge_index[0].reshape(NW, NCH, CH)
    dst = edge_index[1].reshape(NW, NCH, CH)
    eexp, seg = _edge_scores(dst, src, ei, ej)
    parts = _aggregate(dst.reshape(NS * NP, CPP, CH),
                       src.reshape(NS * NP, CPP, CH),
                       eexp.reshape(NS * NP, CPP, CH), seg, h2)
    return _finish(parts)


def kernel(x, edge_index, a_i, a_j, Wx):
    return _impl(x, edge_index, a_i, a_j, Wx)
